# baseline (speedup 1.0000x reference)
import jax
import jax.numpy as jnp
from jax.experimental import pallas as pl


def _copy_body(x_ref, o_ref):
    o_ref[...] = x_ref[...]


def kernel(embeddings):
    return pl.pallas_call(
        _copy_body,
        out_shape=jax.ShapeDtypeStruct(embeddings.shape, embeddings.dtype),
    )(embeddings)
